# baseline (speedup 1.0000x reference)
import functools

import jax
import jax.numpy as jnp
from jax import lax
from jax.experimental import pallas as pl
from jax.experimental.pallas import tpu as pltpu
from jax.experimental.pallas import tpu_sc as plsc

NC = 2
NS = 16
NW = NC * NS
CHUNK = 128
N_PAD = 10240
DUMMY = 10100
MM_BLOCK = 512
DEG_W = 32
K_G = 4
K_GD = 8
STAGE_R = 160
LANES = 16


def _vs_mesh():
    return plsc.VectorSubcoreMesh(
        core_axis_name="c", subcore_axis_name="s", num_cores=NC, num_subcores=NS
    )


@functools.lru_cache(maxsize=None)
def _deg_kernel(k_chunks: int):
    rows = N_PAD // NS
    assert k_chunks % K_GD == 0
    n_grp = k_chunks // K_GD

    def body(dst_hbm, ones_hbm, zero_hbm, out_hbm, acc_sh, dst_v, ones_v, sem):
        c = lax.axis_index("c")
        s = lax.axis_index("s")
        w = s * NC + c
        pltpu.sync_copy(zero_hbm.at[pl.ds(s * rows, rows)],
                        acc_sh.at[pl.ds(s * rows, rows)])
        pltpu.sync_copy(ones_hbm, ones_v)
        pltpu.sync_copy(dst_hbm.at[w], dst_v)
        plsc.subcore_barrier()

        def fire(grp):
            for bb in range(K_GD):
                pltpu.async_copy(ones_v, acc_sh.at[dst_v.at[grp * K_GD + bb]],
                                 sem, add=True)

        fire(0)

        @pl.loop(0, n_grp)
        def _grp(p):
            @pl.when(p < n_grp - 1)
            def _():
                fire(p + 1)

            for bb in range(K_GD):
                pltpu.make_async_copy(
                    zero_hbm.at[pl.ds(0, CHUNK)], ones_v, sem).wait()

        plsc.subcore_barrier()
        pltpu.sync_copy(acc_sh.at[pl.ds(s * rows, rows)],
                        out_hbm.at[c, pl.ds(s * rows, rows)])

    return pl.kernel(
        body,
        out_type=jax.ShapeDtypeStruct((NC, N_PAD, DEG_W), jnp.float32),
        compiler_params=pltpu.CompilerParams(use_tc_tiling_on_sc=False),
        mesh=_vs_mesh(),
        scratch_types=[
            pltpu.VMEM_SHARED((N_PAD, DEG_W), jnp.float32),
            pltpu.VMEM((k_chunks, CHUNK), jnp.int32),
            pltpu.VMEM((CHUNK, DEG_W), jnp.float32),
            pltpu.SemaphoreType.DMA,
        ],
    )


@functools.lru_cache(maxsize=None)
def _hopc_kernel(k_chunks: int, width: int):
    rows = N_PAD // NS
    assert k_chunks % (2 * K_G) == 0
    assert rows % STAGE_R == 0
    n_pairs = k_chunks // (2 * K_G)
    n_stage = rows // STAGE_R

    def body(s_hbm, pa_hbm, pb_hbm, src_hbm, dst_hbm, zero_hbm, out_hbm,
             acc_sh, table_sh, src_v, dst_v, rows_v,
             sv, av, bv, tv, sem_g, sem_s):
        c = lax.axis_index("c")
        s = lax.axis_index("s")
        w = s * NC + c
        pltpu.sync_copy(zero_hbm.at[pl.ds(s * rows, rows)],
                        acc_sh.at[pl.ds(s * rows, rows)])
        pltpu.sync_copy(src_hbm.at[w], src_v)
        pltpu.sync_copy(dst_hbm.at[w], dst_v)

        for q in range(n_stage):
            base = s * rows + q * STAGE_R
            pltpu.sync_copy(s_hbm.at[pl.ds(base, STAGE_R)], sv)
            pltpu.sync_copy(pa_hbm.at[pl.ds(base, STAGE_R)], av)
            pltpu.sync_copy(pb_hbm.at[pl.ds(base, STAGE_R)], bv)

            @pl.loop(0, STAGE_R)
            def _row(r):
                sc = sv[r, pl.ds(0, LANES)]
                for h in range(width // LANES):
                    col = pl.ds(h * LANES, LANES)
                    tv[r, col] = sc * (av[r, col] + bv[r, col])

            pltpu.sync_copy(tv, table_sh.at[pl.ds(base, STAGE_R)])

        plsc.subcore_barrier()

        def fire_g(grp, half):
            for bb in range(K_G):
                pltpu.async_copy(table_sh.at[src_v.at[grp * K_G + bb]],
                                 rows_v.at[half, bb], sem_g)

        def fire_s(grp, half):
            for bb in range(K_G):
                pltpu.async_copy(rows_v.at[half, bb],
                                 acc_sh.at[dst_v.at[grp * K_G + bb]],
                                 sem_s, add=True)

        def drain(sem):
            for bb in range(K_G):
                pltpu.make_async_copy(
                    s_hbm.at[pl.ds(0, CHUNK)], rows_v.at[0, 0], sem).wait()

        fire_g(0, 0)

        @pl.loop(0, n_pairs)
        def _pair(p):
            drain(sem_g)
            fire_g(2 * p + 1, 1)
            fire_s(2 * p, 0)
            drain(sem_s)

            @pl.when(p < n_pairs - 1)
            def _():
                fire_g(2 * p + 2, 0)

            drain(sem_g)
            fire_s(2 * p + 1, 1)
            drain(sem_s)

        plsc.subcore_barrier()
        pltpu.sync_copy(acc_sh.at[pl.ds(s * rows, rows)],
                        out_hbm.at[c, pl.ds(s * rows, rows)])

    return pl.kernel(
        body,
        out_type=jax.ShapeDtypeStruct((NC, N_PAD, width), jnp.float32),
        compiler_params=pltpu.CompilerParams(use_tc_tiling_on_sc=False),
        mesh=_vs_mesh(),
        scratch_types=[
            pltpu.VMEM_SHARED((N_PAD, width), jnp.float32),
            pltpu.VMEM_SHARED((N_PAD, width), jnp.float32),
            pltpu.VMEM((k_chunks, CHUNK), jnp.int32),
            pltpu.VMEM((k_chunks, CHUNK), jnp.int32),
            pltpu.VMEM((2, K_G, CHUNK, width), jnp.float32),
            pltpu.VMEM((STAGE_R, width), jnp.float32),
            pltpu.VMEM((STAGE_R, width), jnp.float32),
            pltpu.VMEM((STAGE_R, width), jnp.float32),
            pltpu.VMEM((STAGE_R, width), jnp.float32),
            pltpu.SemaphoreType.DMA,
            pltpu.SemaphoreType.DMA,
        ],
    )


def _mm_norm(xp, w, degp, n_nodes):
    f = xp.shape[1]
    cdim = w.shape[1]

    def body(x_ref, w_ref, p_ref, y_ref, nb_ref, n2_ref):
        i = pl.program_id(0)
        y = jnp.dot(x_ref[...], w_ref[...], preferred_element_type=jnp.float32,
                    precision=lax.Precision.HIGHEST)
        dsum = p_ref[0, :, :1] + p_ref[1, :, :1]
        row = i * MM_BLOCK + lax.broadcasted_iota(jnp.int32, (MM_BLOCK, 1), 0)
        nrm = jnp.where(row < n_nodes, lax.rsqrt(jnp.maximum(dsum, 1.0)), 0.0)
        y_ref[...] = y
        nb_ref[...] = jnp.broadcast_to(nrm, (MM_BLOCK, cdim))
        n2_ref[...] = jnp.broadcast_to(nrm * nrm, (MM_BLOCK, cdim))

    out = jax.ShapeDtypeStruct((N_PAD, cdim), jnp.float32)
    return pl.pallas_call(
        body,
        grid=(N_PAD // MM_BLOCK,),
        in_specs=[
            pl.BlockSpec((MM_BLOCK, f), lambda i: (i, 0)),
            pl.BlockSpec((f, cdim), lambda i: (0, 0)),
            pl.BlockSpec((NC, MM_BLOCK, DEG_W), lambda i: (0, i, 0)),
        ],
        out_specs=[pl.BlockSpec((MM_BLOCK, cdim), lambda i: (i, 0))] * 3,
        out_shape=[out, out, out],
    )(xp, w, degp)


def _scale(svec, a, c_arr, bias):
    cdim = a.shape[1]

    def body(s_ref, a_ref, c_ref, b_ref, o_ref):
        o_ref[...] = s_ref[...] * (a_ref[...] + c_ref[...]) + b_ref[...]

    return pl.pallas_call(
        body,
        out_shape=jax.ShapeDtypeStruct((N_PAD, cdim), jnp.float32),
    )(svec, a, c_arr, bias)


def kernel(features, edge_index, W, b):
    n, _ = features.shape
    cdim = W.shape[1]
    e = edge_index.shape[1]
    kc_align = 8
    k_chunks = -(-e // (NW * CHUNK))
    k_chunks = -(-k_chunks // kc_align) * kc_align
    e_pad = NW * CHUNK * k_chunks

    src = edge_index[0]
    dst = edge_index[1]
    fill = jnp.full((e_pad - e,), DUMMY, jnp.int32)
    src3 = jnp.concatenate([src, fill]).reshape(NW, k_chunks, CHUNK)
    dst3 = jnp.concatenate([dst, fill]).reshape(NW, k_chunks, CHUNK)

    xp = jnp.pad(features, ((0, N_PAD - n), (0, 0)))
    zeros_w = jnp.zeros((N_PAD, cdim), jnp.float32)
    zeros_d = jnp.zeros((N_PAD, DEG_W), jnp.float32)
    ones_d = jnp.ones((CHUNK, DEG_W), jnp.float32)

    degp = _deg_kernel(k_chunks)(dst3, ones_d, zeros_d)
    y, norm_b, norm2_b = _mm_norm(xp, W, degp, n)
    hop = _hopc_kernel(k_chunks, cdim)
    p1 = hop(norm_b, y, zeros_w, src3, dst3, zeros_w)
    p2 = hop(norm2_b, p1[0], p1[1], src3, dst3, zeros_w)
    out = _scale(norm_b, p2[0], p2[1], jnp.reshape(b, (1, cdim)))
    return out[:n]


# speedup vs baseline: 1.0837x; 1.0837x over previous
import functools

import jax
import jax.numpy as jnp
from jax import lax
from jax.experimental import pallas as pl
from jax.experimental.pallas import tpu as pltpu
from jax.experimental.pallas import tpu_sc as plsc

NC = 2
NS = 16
NW = NC * NS
CHUNK = 128
N_PAD = 10240
DUMMY = 10100
MM_BLOCK = 512
DEG_W = 32
K_G = 4
K_GD = 8


def _vs_mesh():
    return plsc.VectorSubcoreMesh(
        core_axis_name="c", subcore_axis_name="s", num_cores=NC, num_subcores=NS
    )


@functools.lru_cache(maxsize=None)
def _deg_kernel(k_chunks: int):
    rows = N_PAD // NS
    assert k_chunks % K_GD == 0
    n_grp = k_chunks // K_GD

    def body(dst_hbm, ones_hbm, zero_hbm, out_hbm, acc_sh, dst_v, ones_v, sem):
        c = lax.axis_index("c")
        s = lax.axis_index("s")
        w = s * NC + c
        pltpu.sync_copy(zero_hbm.at[pl.ds(s * rows, rows)],
                        acc_sh.at[pl.ds(s * rows, rows)])
        pltpu.sync_copy(ones_hbm, ones_v)
        pltpu.sync_copy(dst_hbm.at[w], dst_v)
        plsc.subcore_barrier()

        def fire(grp):
            for bb in range(K_GD):
                pltpu.async_copy(ones_v, acc_sh.at[dst_v.at[grp * K_GD + bb]],
                                 sem, add=True)

        fire(0)

        @pl.loop(0, n_grp)
        def _grp(p):
            @pl.when(p < n_grp - 1)
            def _():
                fire(p + 1)

            for bb in range(K_GD):
                pltpu.make_async_copy(
                    zero_hbm.at[pl.ds(0, CHUNK)], ones_v, sem).wait()

        plsc.subcore_barrier()
        pltpu.sync_copy(acc_sh.at[pl.ds(s * rows, rows)],
                        out_hbm.at[c, pl.ds(s * rows, rows)])

    return pl.kernel(
        body,
        out_type=jax.ShapeDtypeStruct((NC, N_PAD, DEG_W), jnp.float32),
        compiler_params=pltpu.CompilerParams(use_tc_tiling_on_sc=False),
        mesh=_vs_mesh(),
        scratch_types=[
            pltpu.VMEM_SHARED((N_PAD, DEG_W), jnp.float32),
            pltpu.VMEM((k_chunks, CHUNK), jnp.int32),
            pltpu.VMEM((CHUNK, DEG_W), jnp.float32),
            pltpu.SemaphoreType.DMA,
        ],
    )


@functools.lru_cache(maxsize=None)
def _hop_kernel(k_chunks: int, width: int):
    rows = N_PAD // NS
    assert k_chunks % (2 * K_G) == 0
    n_pairs = k_chunks // (2 * K_G)

    def body(g_hbm, src_hbm, dst_hbm, zero_hbm, out_hbm,
             acc_sh, table_sh, src_v, dst_v, rows_v, sem_g, sem_s):
        c = lax.axis_index("c")
        s = lax.axis_index("s")
        w = s * NC + c
        pltpu.sync_copy(zero_hbm.at[pl.ds(s * rows, rows)],
                        acc_sh.at[pl.ds(s * rows, rows)])
        pltpu.sync_copy(g_hbm.at[pl.ds(s * rows, rows)],
                        table_sh.at[pl.ds(s * rows, rows)])
        pltpu.sync_copy(src_hbm.at[w], src_v)
        pltpu.sync_copy(dst_hbm.at[w], dst_v)
        plsc.subcore_barrier()

        def fire_g(grp, half):
            for bb in range(K_G):
                pltpu.async_copy(table_sh.at[src_v.at[grp * K_G + bb]],
                                 rows_v.at[half, bb], sem_g)

        def fire_s(grp, half):
            for bb in range(K_G):
                pltpu.async_copy(rows_v.at[half, bb],
                                 acc_sh.at[dst_v.at[grp * K_G + bb]],
                                 sem_s, add=True)

        def drain(sem):
            for bb in range(K_G):
                pltpu.make_async_copy(
                    g_hbm.at[pl.ds(0, CHUNK)], rows_v.at[0, 0], sem).wait()

        fire_g(0, 0)

        @pl.loop(0, n_pairs)
        def _pair(p):
            drain(sem_g)
            fire_g(2 * p + 1, 1)
            fire_s(2 * p, 0)
            drain(sem_s)

            @pl.when(p < n_pairs - 1)
            def _():
                fire_g(2 * p + 2, 0)

            drain(sem_g)
            fire_s(2 * p + 1, 1)
            drain(sem_s)

        plsc.subcore_barrier()
        pltpu.sync_copy(acc_sh.at[pl.ds(s * rows, rows)],
                        out_hbm.at[c, pl.ds(s * rows, rows)])

    return pl.kernel(
        body,
        out_type=jax.ShapeDtypeStruct((NC, N_PAD, width), jnp.float32),
        compiler_params=pltpu.CompilerParams(use_tc_tiling_on_sc=False),
        mesh=_vs_mesh(),
        scratch_types=[
            pltpu.VMEM_SHARED((N_PAD, width), jnp.float32),
            pltpu.VMEM_SHARED((N_PAD, width), jnp.float32),
            pltpu.VMEM((k_chunks, CHUNK), jnp.int32),
            pltpu.VMEM((k_chunks, CHUNK), jnp.int32),
            pltpu.VMEM((2, K_G, CHUNK, width), jnp.float32),
            pltpu.SemaphoreType.DMA,
            pltpu.SemaphoreType.DMA,
        ],
    )


def _mm_norm(xp, w, degp, n_nodes):
    f = xp.shape[1]
    cdim = w.shape[1]

    def body(x_ref, w_ref, p_ref, g0_ref, nb_ref, n2_ref):
        i = pl.program_id(0)
        y = jnp.dot(x_ref[...], w_ref[...], preferred_element_type=jnp.float32,
                    precision=lax.Precision.HIGHEST)
        dsum = p_ref[0, :, :1] + p_ref[1, :, :1]
        row = i * MM_BLOCK + lax.broadcasted_iota(jnp.int32, (MM_BLOCK, 1), 0)
        nrm = jnp.where(row < n_nodes, lax.rsqrt(jnp.maximum(dsum, 1.0)), 0.0)
        g0_ref[...] = nrm * y
        nb_ref[...] = jnp.broadcast_to(nrm, (MM_BLOCK, cdim))
        n2_ref[...] = jnp.broadcast_to(nrm * nrm, (MM_BLOCK, cdim))

    out = jax.ShapeDtypeStruct((N_PAD, cdim), jnp.float32)
    return pl.pallas_call(
        body,
        grid=(N_PAD // MM_BLOCK,),
        in_specs=[
            pl.BlockSpec((MM_BLOCK, f), lambda i: (i, 0)),
            pl.BlockSpec((f, cdim), lambda i: (0, 0)),
            pl.BlockSpec((NC, MM_BLOCK, DEG_W), lambda i: (0, i, 0)),
        ],
        out_specs=[pl.BlockSpec((MM_BLOCK, cdim), lambda i: (i, 0))] * 3,
        out_shape=[out, out, out],
    )(xp, w, degp)


def _scale(svec, a, c_arr, bias):
    cdim = a.shape[1]

    def body(s_ref, a_ref, c_ref, b_ref, o_ref):
        o_ref[...] = s_ref[...] * (a_ref[...] + c_ref[...]) + b_ref[...]

    return pl.pallas_call(
        body,
        out_shape=jax.ShapeDtypeStruct((N_PAD, cdim), jnp.float32),
    )(svec, a, c_arr, bias)


def kernel(features, edge_index, W, b):
    n, _ = features.shape
    cdim = W.shape[1]
    e = edge_index.shape[1]
    kc_align = 8
    k_chunks = -(-e // (NW * CHUNK))
    k_chunks = -(-k_chunks // kc_align) * kc_align
    e_pad = NW * CHUNK * k_chunks

    src = edge_index[0]
    dst = edge_index[1]
    fill = jnp.full((e_pad - e,), DUMMY, jnp.int32)
    src3 = jnp.concatenate([src, fill]).reshape(NW, k_chunks, CHUNK)
    dst3 = jnp.concatenate([dst, fill]).reshape(NW, k_chunks, CHUNK)

    xp = jnp.pad(features, ((0, N_PAD - n), (0, 0)))
    zeros_w = jnp.zeros((N_PAD, cdim), jnp.float32)
    zeros_d = jnp.zeros((N_PAD, DEG_W), jnp.float32)
    ones_d = jnp.ones((CHUNK, DEG_W), jnp.float32)

    degp = _deg_kernel(k_chunks)(dst3, ones_d, zeros_d)
    g0, norm_b, norm2_b = _mm_norm(xp, W, degp, n)
    hop = _hop_kernel(k_chunks, cdim)
    p1 = hop(g0, src3, dst3, zeros_w)
    g1 = _scale(norm2_b, p1[0], p1[1], jnp.zeros((1, cdim), jnp.float32))
    p2 = hop(g1, src3, dst3, zeros_w)
    out = _scale(norm_b, p2[0], p2[1], jnp.reshape(b, (1, cdim)))
    return out[:n]


# speedup vs baseline: 1.1138x; 1.0278x over previous
import functools

import jax
import jax.numpy as jnp
from jax import lax
from jax.experimental import pallas as pl
from jax.experimental.pallas import tpu as pltpu
from jax.experimental.pallas import tpu_sc as plsc

NC = 2
NS = 16
NW = NC * NS
CHUNK = 128
N_PAD = 10240
DUMMY = 10100
MM_BLOCK = 512
DEG_W = 32
K_G = 4
K_GD = 8


def _vs_mesh():
    return plsc.VectorSubcoreMesh(
        core_axis_name="c", subcore_axis_name="s", num_cores=NC, num_subcores=NS
    )


@functools.lru_cache(maxsize=None)
def _deg_kernel(k_chunks: int):
    rows = N_PAD // NS
    assert k_chunks % K_GD == 0
    n_grp = k_chunks // K_GD

    def body(dst_hbm, ones_hbm, zero_hbm, out_hbm, acc_sh, dst_v, ones_v, sem):
        c = lax.axis_index("c")
        s = lax.axis_index("s")
        w = s * NC + c
        pltpu.sync_copy(zero_hbm.at[pl.ds(s * rows, rows)],
                        acc_sh.at[pl.ds(s * rows, rows)])
        pltpu.sync_copy(ones_hbm, ones_v)
        pltpu.sync_copy(dst_hbm.at[w], dst_v)
        plsc.subcore_barrier()

        def fire(grp):
            for bb in range(K_GD):
                pltpu.async_copy(ones_v, acc_sh.at[dst_v.at[grp * K_GD + bb]],
                                 sem, add=True)

        fire(0)

        @pl.loop(0, n_grp)
        def _grp(p):
            @pl.when(p < n_grp - 1)
            def _():
                fire(p + 1)

            for bb in range(K_GD):
                pltpu.make_async_copy(
                    zero_hbm.at[pl.ds(0, CHUNK)], ones_v, sem).wait()

        plsc.subcore_barrier()
        pltpu.sync_copy(acc_sh.at[pl.ds(s * rows, rows)],
                        out_hbm.at[c, pl.ds(s * rows, rows)])

    return pl.kernel(
        body,
        out_type=jax.ShapeDtypeStruct((NC, N_PAD, DEG_W), jnp.float32),
        compiler_params=pltpu.CompilerParams(use_tc_tiling_on_sc=False),
        mesh=_vs_mesh(),
        scratch_types=[
            pltpu.VMEM_SHARED((N_PAD, DEG_W), jnp.float32),
            pltpu.VMEM((k_chunks, CHUNK), jnp.int32),
            pltpu.VMEM((CHUNK, DEG_W), jnp.float32),
            pltpu.SemaphoreType.DMA,
        ],
    )


@functools.lru_cache(maxsize=None)
def _hop_kernel(k_chunks: int, width: int):
    rows = N_PAD // NS
    assert k_chunks % (2 * K_G) == 0
    n_pairs = k_chunks // (2 * K_G)

    def body(g_hbm, src_hbm, dst_hbm, zero_hbm, out_hbm,
             acc_sh, table_sh, src_v, dst_v, rows_v, sem_g, sem_s):
        c = lax.axis_index("c")
        s = lax.axis_index("s")
        w = s * NC + c
        pltpu.sync_copy(zero_hbm.at[pl.ds(s * rows, rows)],
                        acc_sh.at[pl.ds(s * rows, rows)])
        pltpu.sync_copy(g_hbm.at[pl.ds(s * rows, rows)],
                        table_sh.at[pl.ds(s * rows, rows)])
        pltpu.sync_copy(src_hbm.at[w], src_v)
        pltpu.sync_copy(dst_hbm.at[w], dst_v)
        plsc.subcore_barrier()

        def fire_g(grp, half):
            for bb in range(K_G):
                pltpu.async_copy(table_sh.at[src_v.at[grp * K_G + bb]],
                                 rows_v.at[half, bb], sem_g)

        def fire_s(grp, half):
            for bb in range(K_G):
                pltpu.async_copy(rows_v.at[half, bb],
                                 acc_sh.at[dst_v.at[grp * K_G + bb]],
                                 sem_s, add=True)

        def drain(sem):
            for bb in range(K_G):
                pltpu.make_async_copy(
                    g_hbm.at[pl.ds(0, CHUNK)], rows_v.at[0, 0], sem).wait()

        fire_g(0, 0)

        @pl.loop(0, n_pairs)
        def _pair(p):
            drain(sem_g)
            fire_g(2 * p + 1, 1)
            fire_s(2 * p, 0)
            drain(sem_s)

            @pl.when(p < n_pairs - 1)
            def _():
                fire_g(2 * p + 2, 0)

            drain(sem_g)
            fire_s(2 * p + 1, 1)
            drain(sem_s)

        plsc.subcore_barrier()
        pltpu.sync_copy(acc_sh.at[pl.ds(s * rows, rows)],
                        out_hbm.at[c, pl.ds(s * rows, rows)])

    return pl.kernel(
        body,
        out_type=jax.ShapeDtypeStruct((NC, N_PAD, width), jnp.float32),
        compiler_params=pltpu.CompilerParams(use_tc_tiling_on_sc=False),
        mesh=_vs_mesh(),
        scratch_types=[
            pltpu.VMEM_SHARED((N_PAD, width), jnp.float32),
            pltpu.VMEM_SHARED((N_PAD, width), jnp.float32),
            pltpu.VMEM((k_chunks, CHUNK), jnp.int32),
            pltpu.VMEM((k_chunks, CHUNK), jnp.int32),
            pltpu.VMEM((2, K_G, CHUNK, width), jnp.float32),
            pltpu.SemaphoreType.DMA,
            pltpu.SemaphoreType.DMA,
        ],
    )


def _mm_norm(xp, w, degp, n_nodes):
    f = xp.shape[1]
    cdim = w.shape[1]

    def body(x_ref, w_ref, p_ref, g0_ref, nb_ref, n2_ref):
        i = pl.program_id(0)
        y = jnp.dot(x_ref[...], w_ref[...], preferred_element_type=jnp.float32,
                    precision=lax.Precision.HIGHEST)
        dsum = p_ref[0, :, :1] + p_ref[1, :, :1]
        row = i * MM_BLOCK + lax.broadcasted_iota(jnp.int32, (MM_BLOCK, 1), 0)
        nrm = jnp.where(row < n_nodes, lax.rsqrt(jnp.maximum(dsum, 1.0)), 0.0)
        g0_ref[...] = nrm * y
        nb_ref[...] = jnp.broadcast_to(nrm, (MM_BLOCK, cdim))
        n2_ref[...] = jnp.broadcast_to(nrm * nrm, (MM_BLOCK, cdim))

    out = jax.ShapeDtypeStruct((N_PAD, cdim), jnp.float32)
    return pl.pallas_call(
        body,
        grid=(N_PAD // MM_BLOCK,),
        in_specs=[
            pl.BlockSpec((MM_BLOCK, f), lambda i: (i, 0)),
            pl.BlockSpec((f, cdim), lambda i: (0, 0)),
            pl.BlockSpec((NC, MM_BLOCK, DEG_W), lambda i: (0, i, 0)),
        ],
        out_specs=[pl.BlockSpec((MM_BLOCK, cdim), lambda i: (i, 0))] * 3,
        out_shape=[out, out, out],
    )(xp, w, degp)


def _pad_edges(edge_index, e_pad):
    e = edge_index.shape[1]

    def body(e_ref, o_ref):
        o_ref[:, :e] = e_ref[...]
        o_ref[:, e:] = jnp.full((2, e_pad - e), DUMMY, jnp.int32)

    return pl.pallas_call(
        body,
        out_shape=jax.ShapeDtypeStruct((2, e_pad), jnp.int32),
    )(edge_index)


def _scale(svec, a, c_arr, bias):
    cdim = a.shape[1]

    def body(s_ref, a_ref, c_ref, b_ref, o_ref):
        o_ref[...] = s_ref[...] * (a_ref[...] + c_ref[...]) + b_ref[...]

    return pl.pallas_call(
        body,
        out_shape=jax.ShapeDtypeStruct((N_PAD, cdim), jnp.float32),
    )(svec, a, c_arr, bias)


def kernel(features, edge_index, W, b):
    n, _ = features.shape
    cdim = W.shape[1]
    e = edge_index.shape[1]
    kc_align = 8
    k_chunks = -(-e // (NW * CHUNK))
    k_chunks = -(-k_chunks // kc_align) * kc_align
    e_pad = NW * CHUNK * k_chunks

    ep = _pad_edges(edge_index, e_pad)
    src3 = ep[0].reshape(NW, k_chunks, CHUNK)
    dst3 = ep[1].reshape(NW, k_chunks, CHUNK)

    xp = jnp.pad(features, ((0, N_PAD - n), (0, 0)))
    zeros_w = jnp.zeros((N_PAD, cdim), jnp.float32)
    zeros_d = jnp.zeros((N_PAD, DEG_W), jnp.float32)
    ones_d = jnp.ones((CHUNK, DEG_W), jnp.float32)

    degp = _deg_kernel(k_chunks)(dst3, ones_d, zeros_d)
    g0, norm_b, norm2_b = _mm_norm(xp, W, degp, n)
    hop = _hop_kernel(k_chunks, cdim)
    p1 = hop(g0, src3, dst3, zeros_w)
    g1 = _scale(norm2_b, p1[0], p1[1], jnp.zeros((1, cdim), jnp.float32))
    p2 = hop(g1, src3, dst3, zeros_w)
    out = _scale(norm_b, p2[0], p2[1], jnp.reshape(b, (1, cdim)))
    return out[:n]


# speedup vs baseline: 1.1366x; 1.0204x over previous
import functools

import jax
import jax.numpy as jnp
from jax import lax
from jax.experimental import pallas as pl
from jax.experimental.pallas import tpu as pltpu
from jax.experimental.pallas import tpu_sc as plsc

NC = 2
NS = 16
NW = NC * NS
CHUNK = 128
N_PAD = 10240
DUMMY = 10100
MM_BLOCK = 512
DEG_W = 32
K_G = 4
K_GD = 8


def _vs_mesh():
    return plsc.VectorSubcoreMesh(
        core_axis_name="c", subcore_axis_name="s", num_cores=NC, num_subcores=NS
    )


@functools.lru_cache(maxsize=None)
def _deg_kernel(k_chunks: int):
    rows = N_PAD // NS
    assert k_chunks % K_GD == 0
    n_grp = k_chunks // K_GD

    def body(dst_hbm, ones_hbm, zero_hbm, out_hbm, acc_sh, dst_v, ones_v, sem):
        c = lax.axis_index("c")
        s = lax.axis_index("s")
        w = s * NC + c
        h1 = pltpu.async_copy(zero_hbm.at[pl.ds(s * rows, rows)],
                              acc_sh.at[pl.ds(s * rows, rows)], sem)
        h2 = pltpu.async_copy(ones_hbm, ones_v, sem)
        h3 = pltpu.async_copy(dst_hbm.at[w], dst_v, sem)
        h1.wait()
        h2.wait()
        h3.wait()
        plsc.subcore_barrier()

        def fire(grp):
            for bb in range(K_GD):
                pltpu.async_copy(ones_v, acc_sh.at[dst_v.at[grp * K_GD + bb]],
                                 sem, add=True)

        fire(0)

        @pl.loop(0, n_grp)
        def _grp(p):
            @pl.when(p < n_grp - 1)
            def _():
                fire(p + 1)

            for bb in range(K_GD):
                pltpu.make_async_copy(
                    zero_hbm.at[pl.ds(0, CHUNK)], ones_v, sem).wait()

        plsc.subcore_barrier()
        pltpu.sync_copy(acc_sh.at[pl.ds(s * rows, rows)],
                        out_hbm.at[c, pl.ds(s * rows, rows)])

    return pl.kernel(
        body,
        out_type=jax.ShapeDtypeStruct((NC, N_PAD, DEG_W), jnp.float32),
        compiler_params=pltpu.CompilerParams(use_tc_tiling_on_sc=False),
        mesh=_vs_mesh(),
        scratch_types=[
            pltpu.VMEM_SHARED((N_PAD, DEG_W), jnp.float32),
            pltpu.VMEM((k_chunks, CHUNK), jnp.int32),
            pltpu.VMEM((CHUNK, DEG_W), jnp.float32),
            pltpu.SemaphoreType.DMA,
        ],
    )


@functools.lru_cache(maxsize=None)
def _hop_kernel(k_chunks: int, width: int):
    rows = N_PAD // NS
    assert k_chunks % (2 * K_G) == 0
    n_pairs = k_chunks // (2 * K_G)

    def body(g_hbm, src_hbm, dst_hbm, zero_hbm, out_hbm,
             acc_sh, table_sh, src_v, dst_v, rows_v, sem_g, sem_s):
        c = lax.axis_index("c")
        s = lax.axis_index("s")
        w = s * NC + c
        h1 = pltpu.async_copy(zero_hbm.at[pl.ds(s * rows, rows)],
                              acc_sh.at[pl.ds(s * rows, rows)], sem_g)
        h2 = pltpu.async_copy(g_hbm.at[pl.ds(s * rows, rows)],
                              table_sh.at[pl.ds(s * rows, rows)], sem_g)
        h3 = pltpu.async_copy(src_hbm.at[w], src_v, sem_g)
        h4 = pltpu.async_copy(dst_hbm.at[w], dst_v, sem_g)
        h1.wait()
        h2.wait()
        h3.wait()
        h4.wait()
        plsc.subcore_barrier()

        def fire_g(grp, half):
            for bb in range(K_G):
                pltpu.async_copy(table_sh.at[src_v.at[grp * K_G + bb]],
                                 rows_v.at[half, bb], sem_g)

        def fire_s(grp, half):
            for bb in range(K_G):
                pltpu.async_copy(rows_v.at[half, bb],
                                 acc_sh.at[dst_v.at[grp * K_G + bb]],
                                 sem_s, add=True)

        def drain(sem):
            for bb in range(K_G):
                pltpu.make_async_copy(
                    g_hbm.at[pl.ds(0, CHUNK)], rows_v.at[0, 0], sem).wait()

        fire_g(0, 0)

        @pl.loop(0, n_pairs)
        def _pair(p):
            drain(sem_g)
            fire_g(2 * p + 1, 1)
            fire_s(2 * p, 0)
            drain(sem_s)

            @pl.when(p < n_pairs - 1)
            def _():
                fire_g(2 * p + 2, 0)

            drain(sem_g)
            fire_s(2 * p + 1, 1)
            drain(sem_s)

        plsc.subcore_barrier()
        pltpu.sync_copy(acc_sh.at[pl.ds(s * rows, rows)],
                        out_hbm.at[c, pl.ds(s * rows, rows)])

    return pl.kernel(
        body,
        out_type=jax.ShapeDtypeStruct((NC, N_PAD, width), jnp.float32),
        compiler_params=pltpu.CompilerParams(use_tc_tiling_on_sc=False),
        mesh=_vs_mesh(),
        scratch_types=[
            pltpu.VMEM_SHARED((N_PAD, width), jnp.float32),
            pltpu.VMEM_SHARED((N_PAD, width), jnp.float32),
            pltpu.VMEM((k_chunks, CHUNK), jnp.int32),
            pltpu.VMEM((k_chunks, CHUNK), jnp.int32),
            pltpu.VMEM((2, K_G, CHUNK, width), jnp.float32),
            pltpu.SemaphoreType.DMA,
            pltpu.SemaphoreType.DMA,
        ],
    )


def _mm_norm(xp, w, degp, n_nodes):
    f = xp.shape[1]
    cdim = w.shape[1]

    def body(x_ref, w_ref, p_ref, g0_ref, nb_ref, n2_ref):
        i = pl.program_id(0)
        y = jnp.dot(x_ref[...], w_ref[...], preferred_element_type=jnp.float32,
                    precision=lax.Precision.HIGHEST)
        dsum = p_ref[0, :, :1] + p_ref[1, :, :1]
        row = i * MM_BLOCK + lax.broadcasted_iota(jnp.int32, (MM_BLOCK, 1), 0)
        nrm = jnp.where(row < n_nodes, lax.rsqrt(jnp.maximum(dsum, 1.0)), 0.0)
        g0_ref[...] = nrm * y
        nb_ref[...] = jnp.broadcast_to(nrm, (MM_BLOCK, cdim))
        n2_ref[...] = jnp.broadcast_to(nrm * nrm, (MM_BLOCK, cdim))

    out = jax.ShapeDtypeStruct((N_PAD, cdim), jnp.float32)
    return pl.pallas_call(
        body,
        grid=(N_PAD // MM_BLOCK,),
        in_specs=[
            pl.BlockSpec((MM_BLOCK, f), lambda i: (i, 0)),
            pl.BlockSpec((f, cdim), lambda i: (0, 0)),
            pl.BlockSpec((NC, MM_BLOCK, DEG_W), lambda i: (0, i, 0)),
        ],
        out_specs=[pl.BlockSpec((MM_BLOCK, cdim), lambda i: (i, 0))] * 3,
        out_shape=[out, out, out],
    )(xp, w, degp)


def _pad_edges(edge_index, e_pad):
    e = edge_index.shape[1]

    def body(e_ref, o_ref):
        o_ref[:, :e] = e_ref[...]
        o_ref[:, e:] = jnp.full((2, e_pad - e), DUMMY, jnp.int32)

    return pl.pallas_call(
        body,
        out_shape=jax.ShapeDtypeStruct((2, e_pad), jnp.int32),
    )(edge_index)


def _scale(svec, a, c_arr, bias):
    cdim = a.shape[1]

    def body(s_ref, a_ref, c_ref, b_ref, o_ref):
        o_ref[...] = s_ref[...] * (a_ref[...] + c_ref[...]) + b_ref[...]

    return pl.pallas_call(
        body,
        out_shape=jax.ShapeDtypeStruct((N_PAD, cdim), jnp.float32),
    )(svec, a, c_arr, bias)


def kernel(features, edge_index, W, b):
    n, _ = features.shape
    cdim = W.shape[1]
    e = edge_index.shape[1]
    kc_align = 8
    k_chunks = -(-e // (NW * CHUNK))
    k_chunks = -(-k_chunks // kc_align) * kc_align
    e_pad = NW * CHUNK * k_chunks

    ep = _pad_edges(edge_index, e_pad)
    src3 = ep[0].reshape(NW, k_chunks, CHUNK)
    dst3 = ep[1].reshape(NW, k_chunks, CHUNK)

    xp = jnp.pad(features, ((0, N_PAD - n), (0, 0)))
    zeros_w = jnp.zeros((N_PAD, cdim), jnp.float32)
    zeros_d = jnp.zeros((N_PAD, DEG_W), jnp.float32)
    ones_d = jnp.ones((CHUNK, DEG_W), jnp.float32)

    degp = _deg_kernel(k_chunks)(dst3, ones_d, zeros_d)
    g0, norm_b, norm2_b = _mm_norm(xp, W, degp, n)
    hop = _hop_kernel(k_chunks, cdim)
    p1 = hop(g0, src3, dst3, zeros_w)
    g1 = _scale(norm2_b, p1[0], p1[1], jnp.zeros((1, cdim), jnp.float32))
    p2 = hop(g1, src3, dst3, zeros_w)
    out = _scale(norm_b, p2[0], p2[1], jnp.reshape(b, (1, cdim)))
    return out[:n]
